# Initial kernel scaffold; baseline (speedup 1.0000x reference)
import jax
import jax.numpy as jnp
from jax.experimental import pallas as pl


def kernel(heads, relations, tails, entity_table, relation_table):
    raise NotImplementedError("write your pallas kernel here")


# speedup vs baseline: 3.6599x; 3.6599x over previous
import functools

import jax
import jax.numpy as jnp
from jax import lax
from jax.experimental import pallas as pl
from jax.experimental.pallas import tpu as pltpu
from jax.experimental.pallas import tpu_sc as plsc

NC = 2
NS = 16
NW = NC * NS
LANES = 16
IW = 128


def _make_embed(B, L, D, CH):
    PW = B // NW
    n_chunks = PW // CH
    ER = CH * 2 * L
    RR = CH * L
    ej = ER // IW
    rj = RR // IW
    EJW = PW * 2 * L // IW
    RJW = PW * L // IW
    nd = D // LANES
    scale = 1.0 / (3.0 * L)

    mesh = plsc.VectorSubcoreMesh(
        core_axis_name="c", subcore_axis_name="s",
        num_cores=NC, num_subcores=NS)

    @functools.partial(
        pl.kernel,
        out_type=jax.ShapeDtypeStruct((B, D), jnp.float32),
        mesh=mesh,
        scratch_types=[
            pltpu.VMEM((EJW, IW), jnp.int32),
            pltpu.VMEM((RJW, IW), jnp.int32),
            pltpu.VMEM((ER, D), jnp.float32),
            pltpu.VMEM((RR, D), jnp.float32),
            pltpu.VMEM((CH, D), jnp.float32),
            pltpu.SemaphoreType.DMA,
        ],
        compiler_params=pltpu.CompilerParams(use_tc_tiling_on_sc=False),
    )
    def embed(ent_idx_hbm, rel_idx_hbm, ent_tab, rel_tab, out_hbm,
              idx_e, idx_r, rows_e, rows_r, outb, sem):
        wid = lax.axis_index("s") * NC + lax.axis_index("c")
        base = wid * PW
        pltpu.sync_copy(ent_idx_hbm.at[pl.ds(wid * EJW, EJW)], idx_e)
        pltpu.sync_copy(rel_idx_hbm.at[pl.ds(wid * RJW, RJW)], idx_r)

        def chunk_body(g, carry):
            off = base + g * CH
            descs = []
            for j in range(ej):
                descs.append(pltpu.async_copy(
                    ent_tab.at[idx_e.at[g * ej + j]],
                    rows_e.at[pl.ds(j * IW, IW)], sem))
            for j in range(rj):
                descs.append(pltpu.async_copy(
                    rel_tab.at[idx_r.at[g * rj + j]],
                    rows_r.at[pl.ds(j * IW, IW)], sem))
            for dsc in descs:
                dsc.wait()

            def path_body(p, carry2):
                pe = p * 2 * L
                pr = p * L
                for d in range(nd):
                    sl = pl.ds(d * LANES, LANES)
                    a0 = rows_e[pe + 0, sl]
                    a1 = rows_e[pe + 1, sl]
                    a2 = rows_e[pe + 2, sl]
                    for j in range(3, 2 * L, 3):
                        a0 = a0 + rows_e[pe + j, sl]
                        if j + 1 < 2 * L:
                            a1 = a1 + rows_e[pe + j + 1, sl]
                        if j + 2 < 2 * L:
                            a2 = a2 + rows_e[pe + j + 2, sl]
                    for j in range(0, L, 3):
                        a0 = a0 + rows_r[pr + j, sl]
                        if j + 1 < L:
                            a1 = a1 + rows_r[pr + j + 1, sl]
                        if j + 2 < L:
                            a2 = a2 + rows_r[pr + j + 2, sl]
                    outb[p, sl] = (a0 + a1 + a2) * scale
                return carry2

            lax.fori_loop(0, CH, path_body, 0)
            pltpu.sync_copy(outb, out_hbm.at[pl.ds(off, CH)])
            return carry

        lax.fori_loop(0, n_chunks, chunk_body, 0)

    return embed


def kernel(heads, relations, tails, entity_table, relation_table):
    B, L = heads.shape
    D = entity_table.shape[1]
    ent_idx = jnp.concatenate(
        [heads.astype(jnp.int32), tails.astype(jnp.int32)], axis=1)
    ent_idx = ent_idx.reshape(B * 2 * L // IW, IW)
    rel_idx = relations.astype(jnp.int32).reshape(B * L // IW, IW)
    embed = _make_embed(B, L, D, CH=64)
    return embed(ent_idx, rel_idx, entity_table, relation_table)


# speedup vs baseline: 4.0792x; 1.1146x over previous
import functools

import jax
import jax.numpy as jnp
from jax import lax
from jax.experimental import pallas as pl
from jax.experimental.pallas import tpu as pltpu
from jax.experimental.pallas import tpu_sc as plsc

NC = 2
NS = 16
NW = NC * NS
LANES = 16
IW = 128


def _make_embed(B, L, D, CH):
    PW = B // NW
    n_chunks = PW // CH
    assert n_chunks % 2 == 0
    RPC = CH * L
    jc = RPC // IW
    JW = PW * L // IW
    nd = D // LANES
    scale = 1.0 / (3.0 * L)

    mesh = plsc.VectorSubcoreMesh(
        core_axis_name="c", subcore_axis_name="s",
        num_cores=NC, num_subcores=NS)

    @functools.partial(
        pl.kernel,
        out_type=jax.ShapeDtypeStruct((B, D), jnp.float32),
        mesh=mesh,
        scratch_types=[
            pltpu.VMEM((JW, IW), jnp.int32),
            pltpu.VMEM((JW, IW), jnp.int32),
            pltpu.VMEM((JW, IW), jnp.int32),
            pltpu.VMEM((2, 2 * RPC, D), jnp.float32),
            pltpu.VMEM((2, RPC, D), jnp.float32),
            pltpu.VMEM((2, CH, D), jnp.float32),
            pltpu.SemaphoreType.DMA,
            pltpu.SemaphoreType.DMA,
        ],
        compiler_params=pltpu.CompilerParams(use_tc_tiling_on_sc=False),
    )
    def embed(hid_hbm, tid_hbm, rid_hbm, ent_tab, rel_tab, out_hbm,
              idx_h, idx_t, idx_r, rows_e, rows_r, outb, sem0, sem1):
        sems = (sem0, sem1)
        wid = lax.axis_index("s") * NC + lax.axis_index("c")
        base = wid * PW
        pltpu.sync_copy(hid_hbm.at[pl.ds(wid * JW, JW)], idx_h)
        pltpu.sync_copy(tid_hbm.at[pl.ds(wid * JW, JW)], idx_t)
        pltpu.sync_copy(rid_hbm.at[pl.ds(wid * JW, JW)], idx_r)

        def fire(g, s):
            for j in range(jc):
                pltpu.async_copy(
                    ent_tab.at[idx_h.at[g * jc + j]],
                    rows_e.at[s, pl.ds(j * IW, IW)], sems[s])
                pltpu.async_copy(
                    ent_tab.at[idx_t.at[g * jc + j]],
                    rows_e.at[s, pl.ds(RPC + j * IW, IW)], sems[s])
                pltpu.async_copy(
                    rel_tab.at[idx_r.at[g * jc + j]],
                    rows_r.at[s, pl.ds(j * IW, IW)], sems[s])

        def drain(s):
            for j in range(jc):
                pltpu.make_async_copy(
                    ent_tab.at[idx_h.at[0]],
                    rows_e.at[s, pl.ds(j * IW, IW)], sems[s]).wait()
                pltpu.make_async_copy(
                    ent_tab.at[idx_t.at[0]],
                    rows_e.at[s, pl.ds(RPC + j * IW, IW)], sems[s]).wait()
                pltpu.make_async_copy(
                    rel_tab.at[idx_r.at[0]],
                    rows_r.at[s, pl.ds(j * IW, IW)], sems[s]).wait()

        def compute(g, s):
            def path_body(p, carry2):
                ph = p * L
                pt = RPC + p * L
                for d in range(nd):
                    sl = pl.ds(d * LANES, LANES)
                    a0 = rows_e[s, ph + 0, sl]
                    a1 = rows_e[s, ph + 1, sl]
                    a2 = rows_e[s, ph + 2, sl]
                    for j in range(3, L, 3):
                        a0 = a0 + rows_e[s, ph + j, sl]
                        if j + 1 < L:
                            a1 = a1 + rows_e[s, ph + j + 1, sl]
                        if j + 2 < L:
                            a2 = a2 + rows_e[s, ph + j + 2, sl]
                    for j in range(0, L, 3):
                        a0 = a0 + rows_e[s, pt + j, sl]
                        if j + 1 < L:
                            a1 = a1 + rows_e[s, pt + j + 1, sl]
                        if j + 2 < L:
                            a2 = a2 + rows_e[s, pt + j + 2, sl]
                    for j in range(0, L, 3):
                        a0 = a0 + rows_r[s, p * L + j, sl]
                        if j + 1 < L:
                            a1 = a1 + rows_r[s, p * L + j + 1, sl]
                        if j + 2 < L:
                            a2 = a2 + rows_r[s, p * L + j + 2, sl]
                    outb[s, p, sl] = (a0 + a1 + a2) * scale
                return carry2

            lax.fori_loop(0, CH, path_body, 0)
            pltpu.sync_copy(outb.at[s], out_hbm.at[pl.ds(base + g * CH, CH)])

        fire(0, 0)

        def pair_body(k, carry):
            g0 = k * 2
            fire(g0 + 1, 1)
            drain(0)
            compute(g0, 0)

            @pl.when(g0 + 2 < n_chunks)
            def _():
                fire(g0 + 2, 0)

            drain(1)
            compute(g0 + 1, 1)
            return carry

        lax.fori_loop(0, n_chunks // 2, pair_body, 0)

    return embed


def kernel(heads, relations, tails, entity_table, relation_table):
    B, L = heads.shape
    D = entity_table.shape[1]
    hid = heads.astype(jnp.int32).reshape(B * L // IW, IW)
    tid = tails.astype(jnp.int32).reshape(B * L // IW, IW)
    rid = relations.astype(jnp.int32).reshape(B * L // IW, IW)
    embed = _make_embed(B, L, D, CH=32)
    return embed(hid, tid, rid, entity_table, relation_table)


# speedup vs baseline: 4.2650x; 1.0456x over previous
import functools

import jax
import jax.numpy as jnp
from jax import lax
from jax.experimental import pallas as pl
from jax.experimental.pallas import tpu as pltpu
from jax.experimental.pallas import tpu_sc as plsc

NC = 2
NS = 16
NW = NC * NS
LANES = 16
PB = 512


def _make_embed(B, L, D, V, R):
    DPW = D // NW
    NB = B // PB
    assert NB % 2 == 0
    NG = PB // LANES
    scale = 1.0 / (3.0 * L)

    mesh = plsc.VectorSubcoreMesh(
        core_axis_name="c", subcore_axis_name="s",
        num_cores=NC, num_subcores=NS)

    @functools.partial(
        pl.kernel,
        out_type=jax.ShapeDtypeStruct((D, B), jnp.float32),
        mesh=mesh,
        scratch_types=[
            pltpu.VMEM((V,), jnp.float32),
            pltpu.VMEM((DPW, R), jnp.float32),
            pltpu.VMEM((2, 3 * L, PB), jnp.int32),
            pltpu.VMEM((2, PB), jnp.float32),
            pltpu.SemaphoreType.DMA,
            pltpu.SemaphoreType.DMA,
        ],
        compiler_params=pltpu.CompilerParams(
            use_tc_tiling_on_sc=False, needs_layout_passes=False),
    )
    def embed(h_t, t_t, r_t, ent_t, rel_t, out_t,
              ent_row, rel_rows, idx_buf, outb, sem0, sem1):
        sems = (sem0, sem1)
        wid = lax.axis_index("s") * NC + lax.axis_index("c")
        pltpu.sync_copy(rel_t.at[pl.ds(wid * DPW, DPW)], rel_rows)

        def fire(g, s):
            off = g * PB
            pltpu.async_copy(h_t.at[:, pl.ds(off, PB)],
                             idx_buf.at[s, pl.ds(0, L)], sems[s])
            pltpu.async_copy(t_t.at[:, pl.ds(off, PB)],
                             idx_buf.at[s, pl.ds(L, L)], sems[s])
            pltpu.async_copy(r_t.at[:, pl.ds(off, PB)],
                             idx_buf.at[s, pl.ds(2 * L, L)], sems[s])

        def drain(s):
            for o in range(3):
                pltpu.make_async_copy(
                    h_t.at[:, pl.ds(0, PB)],
                    idx_buf.at[s, pl.ds(o * L, L)], sems[s]).wait()

        for p in range(DPW):
            d = wid * DPW + p
            pltpu.sync_copy(ent_t.at[d], ent_row)
            rel_row = rel_rows.at[p]
            fire(0, 0)

            def compute(g, s):
                def group_body(gg, carry2):
                    sl = pl.ds(gg * LANES, LANES)
                    a0 = plsc.load_gather(ent_row, [idx_buf[s, 0, sl]])
                    a1 = plsc.load_gather(ent_row, [idx_buf[s, 1, sl]])
                    a2 = plsc.load_gather(ent_row, [idx_buf[s, 2, sl]])
                    for j in range(3, 2 * L):
                        v = plsc.load_gather(ent_row, [idx_buf[s, j, sl]])
                        if j % 3 == 0:
                            a0 = a0 + v
                        elif j % 3 == 1:
                            a1 = a1 + v
                        else:
                            a2 = a2 + v
                    for j in range(2 * L, 3 * L):
                        v = plsc.load_gather(rel_row, [idx_buf[s, j, sl]])
                        if j % 3 == 0:
                            a0 = a0 + v
                        elif j % 3 == 1:
                            a1 = a1 + v
                        else:
                            a2 = a2 + v
                    outb[s, sl] = (a0 + a1 + a2) * scale
                    return carry2

                lax.fori_loop(0, NG, group_body, 0)
                pltpu.sync_copy(outb.at[s],
                                out_t.at[d, pl.ds(g * PB, PB)])

            def pair_body(k, carry):
                g0 = k * 2
                fire(g0 + 1, 1)
                drain(0)
                compute(g0, 0)

                @pl.when(g0 + 2 < NB)
                def _():
                    fire(g0 + 2, 0)

                drain(1)
                compute(g0 + 1, 1)
                return carry

            lax.fori_loop(0, NB // 2, pair_body, 0)

    return embed


def kernel(heads, relations, tails, entity_table, relation_table):
    B, L = heads.shape
    V, D = entity_table.shape
    R = relation_table.shape[0]
    embed = _make_embed(B, L, D, V, R)
    out_t = embed(heads.astype(jnp.int32).T,
                  tails.astype(jnp.int32).T,
                  relations.astype(jnp.int32).T,
                  entity_table.T, relation_table.T)
    return out_t.T


# speedup vs baseline: 4.8567x; 1.1387x over previous
import functools

import jax
import jax.numpy as jnp
from jax import lax
from jax.experimental import pallas as pl
from jax.experimental.pallas import tpu as pltpu
from jax.experimental.pallas import tpu_sc as plsc

NC = 2
NS = 16
NW = NC * NS
LANES = 16
PB = 512


def _make_embed(B, L, D, V, R):
    DPW = D // NW
    NB = B // PB
    assert NB % 2 == 0
    NG = PB // LANES
    scale = 1.0 / (3.0 * L)

    mesh = plsc.VectorSubcoreMesh(
        core_axis_name="c", subcore_axis_name="s",
        num_cores=NC, num_subcores=NS)

    @functools.partial(
        pl.kernel,
        out_type=jax.ShapeDtypeStruct((D, B), jnp.float32),
        mesh=mesh,
        scratch_types=[
            pltpu.VMEM((V,), jnp.float32),
            pltpu.VMEM((DPW, R), jnp.float32),
            pltpu.VMEM((2, 3 * L, PB), jnp.int32),
            pltpu.VMEM((2, PB), jnp.float32),
            pltpu.SemaphoreType.DMA,
            pltpu.SemaphoreType.DMA,
            pltpu.SemaphoreType.DMA,
            pltpu.SemaphoreType.DMA,
        ],
        compiler_params=pltpu.CompilerParams(
            use_tc_tiling_on_sc=False, needs_layout_passes=False,
            disable_bounds_checks=True),
    )
    def embed(h_t, t_t, r_t, ent_t, rel_t, out_t,
              ent_row, rel_rows, idx_buf, outb, sem0, sem1, sem_o0, sem_o1):
        sems = (sem0, sem1)
        wid = lax.axis_index("s") * NC + lax.axis_index("c")
        pltpu.sync_copy(rel_t.at[pl.ds(wid * DPW, DPW)], rel_rows)

        def fire(g, s):
            off = g * PB
            pltpu.async_copy(h_t.at[:, pl.ds(off, PB)],
                             idx_buf.at[s, pl.ds(0, L)], sems[s])
            pltpu.async_copy(t_t.at[:, pl.ds(off, PB)],
                             idx_buf.at[s, pl.ds(L, L)], sems[s])
            pltpu.async_copy(r_t.at[:, pl.ds(off, PB)],
                             idx_buf.at[s, pl.ds(2 * L, L)], sems[s])

        def drain(s):
            for o in range(3):
                pltpu.make_async_copy(
                    h_t.at[:, pl.ds(0, PB)],
                    idx_buf.at[s, pl.ds(o * L, L)], sems[s]).wait()

        out_sems = (sem_o0, sem_o1)

        def wait_out(s):
            pltpu.make_async_copy(
                outb.at[s], out_t.at[0, pl.ds(0, PB)], out_sems[s]).wait()

        for p in range(DPW):
            d = wid * DPW + p
            pltpu.sync_copy(ent_t.at[d], ent_row)
            rel_row = rel_rows.at[p]
            fire(0, 0)

            def compute(g, s, first):
                if not first:
                    wait_out(s)

                @functools.partial(plsc.parallel_loop, 0, NG, unroll=2)
                def group_body(gg):
                    sl = pl.ds(gg * LANES, LANES)
                    a0 = plsc.load_gather(ent_row, [idx_buf[s, 0, sl]])
                    a1 = plsc.load_gather(ent_row, [idx_buf[s, 1, sl]])
                    a2 = plsc.load_gather(ent_row, [idx_buf[s, 2, sl]])
                    for j in range(3, 2 * L):
                        v = plsc.load_gather(ent_row, [idx_buf[s, j, sl]])
                        if j % 3 == 0:
                            a0 = a0 + v
                        elif j % 3 == 1:
                            a1 = a1 + v
                        else:
                            a2 = a2 + v
                    for j in range(2 * L, 3 * L):
                        v = plsc.load_gather(rel_row, [idx_buf[s, j, sl]])
                        if j % 3 == 0:
                            a0 = a0 + v
                        elif j % 3 == 1:
                            a1 = a1 + v
                        else:
                            a2 = a2 + v
                    outb[s, sl] = (a0 + a1 + a2) * scale

                pltpu.async_copy(outb.at[s],
                                 out_t.at[d, pl.ds(g * PB, PB)], out_sems[s])

            fire(1, 1)
            drain(0)
            compute(0, 0, True)
            fire(2, 0)
            drain(1)
            compute(1, 1, True)

            def pair_body(k, carry):
                g0 = k * 2
                fire(g0 + 1, 1)
                drain(0)
                compute(g0, 0, False)

                @pl.when(g0 + 2 < NB)
                def _():
                    fire(g0 + 2, 0)

                drain(1)
                compute(g0 + 1, 1, False)
                return carry

            lax.fori_loop(1, NB // 2, pair_body, 0)
            wait_out(0)
            wait_out(1)

    return embed


def kernel(heads, relations, tails, entity_table, relation_table):
    B, L = heads.shape
    V, D = entity_table.shape
    R = relation_table.shape[0]
    embed = _make_embed(B, L, D, V, R)
    out_t = embed(heads.astype(jnp.int32).T,
                  tails.astype(jnp.int32).T,
                  relations.astype(jnp.int32).T,
                  entity_table.T, relation_table.T)
    return out_t.T
